# baseline (device time: 34509 ns/iter reference)
import jax
import jax.numpy as jnp
from jax import lax
from jax.experimental import pallas as pl
from jax.experimental.pallas import tpu as pltpu

N_DEV = 8
B = 2
SQ = 256
SKV = 256
HQ_LOCAL = 4
DH = 64
D_MODEL = 512
CHUNK = HQ_LOCAL * DH
ROWS = B * SQ
SCALE = 0.125 * 1.4426950408889634


def kernel(x, Wq, K_ext, V_ext, Wo):
    def body(x_ref, wq_ref, k_ref, v_ref, wo_ref, out_ref,
             x_v, wq_v, k_v, v_v, wo_v, mychunk_ref, gather_ref,
             copy_sems, send_sems, recv_sems):
        my = lax.axis_index("i")

        cp_x = pltpu.make_async_copy(x_ref, x_v, copy_sems.at[0])
        cp_wq = pltpu.make_async_copy(
            wq_ref.at[:, pl.ds(my * CHUNK, CHUNK)], wq_v, copy_sems.at[1])
        cp_k = pltpu.make_async_copy(k_ref, k_v, copy_sems.at[2])
        cp_v = pltpu.make_async_copy(v_ref, v_v, copy_sems.at[3])
        cp_wo = pltpu.make_async_copy(wo_ref, wo_v, copy_sems.at[4])
        cp_x.start()
        cp_wq.start()
        cp_k.start()
        cp_v.start()
        cp_wo.start()

        barrier_sem = pltpu.get_barrier_semaphore()
        for d in range(1, N_DEV):
            pl.semaphore_signal(
                barrier_sem, inc=1,
                device_id=(lax.rem(my + d, N_DEV),),
                device_id_type=pl.DeviceIdType.MESH,
            )
        pl.semaphore_wait(barrier_sem, N_DEV - 1)

        cp_x.wait()
        cp_wq.wait()
        x2d = x_v[...].reshape(ROWS, D_MODEL).astype(jnp.bfloat16)
        q2d = jnp.dot(x2d, wq_v[...].astype(jnp.bfloat16),
                      preferred_element_type=jnp.float32) * SCALE

        qb = lax.broadcasted_iota(jnp.int32, (SQ, SKV), 0) // 64
        kb = lax.broadcasted_iota(jnp.int32, (SQ, SKV), 1) // 64
        mask = (qb == kb) | ((kb % 4) == (qb % 4))

        cp_k.wait()
        cp_v.wait()
        for b in range(B):
            for h in range(HQ_LOCAL):
                q_bh = q2d[b * SQ:(b + 1) * SQ,
                           h * DH:(h + 1) * DH].astype(jnp.bfloat16)
                k_bh = k_v[b, :, h, :].astype(jnp.bfloat16)
                s = lax.dot_general(
                    q_bh, k_bh, (((1,), (1,)), ((), ())),
                    preferred_element_type=jnp.float32)
                w = jnp.exp2(jnp.where(mask, s, -1e9))
                w = w * (1.0 / jnp.sum(w, axis=-1, keepdims=True))
                v_bh = v_v[b, :, h, :].astype(jnp.bfloat16)
                ctx_bh = jnp.dot(w.astype(jnp.bfloat16), v_bh,
                                 preferred_element_type=jnp.float32)
                mychunk_ref[b * SQ:(b + 1) * SQ,
                            h * DH:(h + 1) * DH] = ctx_bh.astype(jnp.bfloat16)

        rdmas = []
        for d in range(1, N_DEV):
            rdma = pltpu.make_async_remote_copy(
                src_ref=mychunk_ref,
                dst_ref=gather_ref.at[my],
                send_sem=send_sems.at[d - 1],
                recv_sem=recv_sems.at[d - 1],
                device_id=(lax.rem(my + d, N_DEV),),
                device_id_type=pl.DeviceIdType.MESH,
            )
            rdma.start()
            rdmas.append(rdma)

        cp_wo.wait()
        wo_my = wo_v[pl.ds(my * CHUNK, CHUNK), :].astype(jnp.bfloat16)
        acc = jnp.dot(mychunk_ref[...], wo_my,
                      preferred_element_type=jnp.float32)

        for d in range(1, N_DEV):
            rdmas[d - 1].wait_recv()
            origin = lax.rem(my - d + N_DEV, N_DEV)
            wo_o = wo_v[pl.ds(origin * CHUNK, CHUNK), :].astype(jnp.bfloat16)
            acc = acc + jnp.dot(gather_ref[origin], wo_o,
                                preferred_element_type=jnp.float32)
        out_ref[...] = acc.astype(jnp.bfloat16).reshape(B, SQ, D_MODEL)

        for d in range(1, N_DEV):
            rdmas[d - 1].wait_send()

    return pl.pallas_call(
        body,
        out_shape=jax.ShapeDtypeStruct((B, SQ, D_MODEL), jnp.bfloat16),
        in_specs=[
            pl.BlockSpec(memory_space=pltpu.MemorySpace.HBM),
            pl.BlockSpec(memory_space=pltpu.MemorySpace.HBM),
            pl.BlockSpec(memory_space=pltpu.MemorySpace.HBM),
            pl.BlockSpec(memory_space=pltpu.MemorySpace.HBM),
            pl.BlockSpec(memory_space=pltpu.MemorySpace.HBM),
        ],
        out_specs=pl.BlockSpec(memory_space=pltpu.VMEM),
        scratch_shapes=[
            pltpu.VMEM((B, SQ, D_MODEL), jnp.float32),
            pltpu.VMEM((D_MODEL, CHUNK), jnp.float32),
            pltpu.VMEM((B, SKV, HQ_LOCAL, DH), jnp.float32),
            pltpu.VMEM((B, SKV, HQ_LOCAL, DH), jnp.float32),
            pltpu.VMEM((HQ_LOCAL * DH * N_DEV, D_MODEL), jnp.float32),
            pltpu.VMEM((ROWS, CHUNK), jnp.bfloat16),
            pltpu.VMEM((N_DEV, ROWS, CHUNK), jnp.bfloat16),
            pltpu.SemaphoreType.DMA((5,)),
            pltpu.SemaphoreType.DMA((N_DEV - 1,)),
            pltpu.SemaphoreType.DMA((N_DEV - 1,)),
        ],
        compiler_params=pltpu.CompilerParams(collective_id=0),
    )(x, Wq, K_ext, V_ext, Wo)


# device time: 30955 ns/iter; 1.1148x vs baseline; 1.1148x over previous
import jax
import jax.numpy as jnp
from jax import lax
from jax.experimental import pallas as pl
from jax.experimental.pallas import tpu as pltpu

N_DEV = 8
B = 2
SQ = 256
SKV = 256
HQ_LOCAL = 4
DH = 64
D_MODEL = 512
CHUNK = HQ_LOCAL * DH
ROWS = B * SQ
SCALE = 0.125 * 1.4426950408889634


def kernel(x, Wq, K_ext, V_ext, Wo):
    def body(x_ref, wq_ref, k_ref, v_ref, wo_ref, out_ref,
             x_v, wq_v, k_v, v_v, wo_v, mychunk_ref, gather_ref,
             copy_sems, send_sems, recv_sems):
        my = lax.axis_index("i")

        cp_x = pltpu.make_async_copy(x_ref, x_v, copy_sems.at[0])
        cp_wq = pltpu.make_async_copy(
            wq_ref.at[:, pl.ds(my * CHUNK, CHUNK)], wq_v, copy_sems.at[1])
        cp_k = pltpu.make_async_copy(k_ref, k_v, copy_sems.at[2])
        cp_v = pltpu.make_async_copy(v_ref, v_v, copy_sems.at[3])
        cp_wo = pltpu.make_async_copy(wo_ref, wo_v, copy_sems.at[4])
        cp_x.start()
        cp_wq.start()
        cp_k.start()
        cp_v.start()
        cp_wo.start()

        cp_x.wait()
        cp_wq.wait()
        x2d = x_v[...].reshape(ROWS, D_MODEL).astype(jnp.bfloat16)
        q2d = jnp.dot(x2d, wq_v[...].astype(jnp.bfloat16),
                      preferred_element_type=jnp.float32) * SCALE

        qb = lax.broadcasted_iota(jnp.int32, (SQ, SKV), 0) // 64
        kb = lax.broadcasted_iota(jnp.int32, (SQ, SKV), 1) // 64
        mask = (qb == kb) | ((kb % 4) == (qb % 4))

        cp_k.wait()
        cp_v.wait()

        def attn_batch(b):
            for h in range(HQ_LOCAL):
                q_bh = q2d[b * SQ:(b + 1) * SQ,
                           h * DH:(h + 1) * DH].astype(jnp.bfloat16)
                k_bh = k_v[b, :, h, :].astype(jnp.bfloat16)
                s = lax.dot_general(
                    q_bh, k_bh, (((1,), (1,)), ((), ())),
                    preferred_element_type=jnp.float32)
                w = jnp.exp2(jnp.where(mask, s, -1e9))
                w = w * (1.0 / jnp.sum(w, axis=-1, keepdims=True))
                v_bh = v_v[b, :, h, :].astype(jnp.bfloat16)
                ctx_bh = jnp.dot(w.astype(jnp.bfloat16), v_bh,
                                 preferred_element_type=jnp.float32)
                mychunk_ref[b * SQ:(b + 1) * SQ,
                            h * DH:(h + 1) * DH] = ctx_bh.astype(jnp.bfloat16)

        def send_wave(b):
            wave = []
            for d in range(1, N_DEV):
                rdma = pltpu.make_async_remote_copy(
                    src_ref=mychunk_ref.at[pl.ds(b * SQ, SQ)],
                    dst_ref=gather_ref.at[my, pl.ds(b * SQ, SQ)],
                    send_sem=send_sems.at[d - 1, b],
                    recv_sem=recv_sems.at[d - 1, b],
                    device_id=(lax.rem(my + d, N_DEV),),
                    device_id_type=pl.DeviceIdType.MESH,
                )
                rdma.start()
                wave.append(rdma)
            return wave

        attn_batch(0)

        barrier_sem = pltpu.get_barrier_semaphore()
        for d in range(1, N_DEV):
            pl.semaphore_signal(
                barrier_sem, inc=1,
                device_id=(lax.rem(my + d, N_DEV),),
                device_id_type=pl.DeviceIdType.MESH,
            )
        pl.semaphore_wait(barrier_sem, N_DEV - 1)

        wave0 = send_wave(0)
        attn_batch(1)
        wave1 = send_wave(1)

        cp_wo.wait()
        wo_my = wo_v[pl.ds(my * CHUNK, CHUNK), :].astype(jnp.bfloat16)
        acc = jnp.dot(mychunk_ref[...], wo_my,
                      preferred_element_type=jnp.float32)

        for d in range(1, N_DEV):
            wave0[d - 1].wait_recv()
            wave1[d - 1].wait_recv()
            origin = lax.rem(my - d + N_DEV, N_DEV)
            wo_o = wo_v[pl.ds(origin * CHUNK, CHUNK), :].astype(jnp.bfloat16)
            acc = acc + jnp.dot(gather_ref[origin], wo_o,
                                preferred_element_type=jnp.float32)
        out_ref[...] = acc.astype(jnp.bfloat16).reshape(B, SQ, D_MODEL)

        for d in range(1, N_DEV):
            wave0[d - 1].wait_send()
            wave1[d - 1].wait_send()

    return pl.pallas_call(
        body,
        out_shape=jax.ShapeDtypeStruct((B, SQ, D_MODEL), jnp.bfloat16),
        in_specs=[
            pl.BlockSpec(memory_space=pltpu.MemorySpace.HBM),
            pl.BlockSpec(memory_space=pltpu.MemorySpace.HBM),
            pl.BlockSpec(memory_space=pltpu.MemorySpace.HBM),
            pl.BlockSpec(memory_space=pltpu.MemorySpace.HBM),
            pl.BlockSpec(memory_space=pltpu.MemorySpace.HBM),
        ],
        out_specs=pl.BlockSpec(memory_space=pltpu.VMEM),
        scratch_shapes=[
            pltpu.VMEM((B, SQ, D_MODEL), jnp.float32),
            pltpu.VMEM((D_MODEL, CHUNK), jnp.float32),
            pltpu.VMEM((B, SKV, HQ_LOCAL, DH), jnp.float32),
            pltpu.VMEM((B, SKV, HQ_LOCAL, DH), jnp.float32),
            pltpu.VMEM((HQ_LOCAL * DH * N_DEV, D_MODEL), jnp.float32),
            pltpu.VMEM((ROWS, CHUNK), jnp.bfloat16),
            pltpu.VMEM((N_DEV, ROWS, CHUNK), jnp.bfloat16),
            pltpu.SemaphoreType.DMA((5,)),
            pltpu.SemaphoreType.DMA((N_DEV - 1, B)),
            pltpu.SemaphoreType.DMA((N_DEV - 1, B)),
        ],
        compiler_params=pltpu.CompilerParams(collective_id=0),
    )(x, Wq, K_ext, V_ext, Wo)


# device time: 25192 ns/iter; 1.3698x vs baseline; 1.2288x over previous
import jax
import jax.numpy as jnp
from jax import lax
from jax.experimental import pallas as pl
from jax.experimental.pallas import tpu as pltpu

N_DEV = 8
B = 2
SQ = 256
SKV = 256
HQ_LOCAL = 4
DH = 64
D_MODEL = 512
CHUNK = HQ_LOCAL * DH
ROWS = B * SQ
SCALE = 0.125 * 1.4426950408889634


def kernel(x, Wq, K_ext, V_ext, Wo):
    idx = lax.axis_index("i")
    x_b = x.astype(jnp.bfloat16)
    wq_b = lax.dynamic_slice_in_dim(Wq, idx * CHUNK, CHUNK, 1).astype(
        jnp.bfloat16)
    k_b = K_ext.astype(jnp.bfloat16)
    v_b = V_ext.astype(jnp.bfloat16)
    wo_b = Wo.astype(jnp.bfloat16)

    def body(x_ref, wq_ref, k_ref, v_ref, wo_ref, out_ref,
             x_v, wq_v, k_v, v_v, wo_v, mychunk_ref, gather_ref,
             copy_sems, send_sems, recv_sems):
        my = lax.axis_index("i")

        cp_x = pltpu.make_async_copy(x_ref, x_v, copy_sems.at[0])
        cp_wq = pltpu.make_async_copy(wq_ref, wq_v, copy_sems.at[1])
        cp_k = pltpu.make_async_copy(k_ref, k_v, copy_sems.at[2])
        cp_v = pltpu.make_async_copy(v_ref, v_v, copy_sems.at[3])
        cp_wo = pltpu.make_async_copy(wo_ref, wo_v, copy_sems.at[4])
        cp_x.start()
        cp_wq.start()
        cp_k.start()
        cp_v.start()
        cp_wo.start()

        cp_x.wait()
        cp_wq.wait()
        x2d = x_v[...].reshape(ROWS, D_MODEL)
        q2d = jnp.dot(x2d, wq_v[...],
                      preferred_element_type=jnp.float32) * SCALE

        qb = lax.broadcasted_iota(jnp.int32, (SQ, SKV), 0) // 64
        kb = lax.broadcasted_iota(jnp.int32, (SQ, SKV), 1) // 64
        mask = (qb == kb) | ((kb % 4) == (qb % 4))

        cp_k.wait()
        cp_v.wait()

        def attn_batch(b):
            for h in range(HQ_LOCAL):
                q_bh = q2d[b * SQ:(b + 1) * SQ,
                           h * DH:(h + 1) * DH].astype(jnp.bfloat16)
                k_bh = k_v[b, :, h, :]
                s = lax.dot_general(
                    q_bh, k_bh, (((1,), (1,)), ((), ())),
                    preferred_element_type=jnp.float32)
                w = jnp.exp2(jnp.where(mask, s, -1e9))
                w = w * (1.0 / jnp.sum(w, axis=-1, keepdims=True))
                ctx_bh = jnp.dot(w.astype(jnp.bfloat16), v_v[b, :, h, :],
                                 preferred_element_type=jnp.float32)
                mychunk_ref[b * SQ:(b + 1) * SQ,
                            h * DH:(h + 1) * DH] = ctx_bh.astype(jnp.bfloat16)

        def send_wave(b):
            wave = []
            for d in range(1, N_DEV):
                rdma = pltpu.make_async_remote_copy(
                    src_ref=mychunk_ref.at[pl.ds(b * SQ, SQ)],
                    dst_ref=gather_ref.at[my, pl.ds(b * SQ, SQ)],
                    send_sem=send_sems.at[d - 1, b],
                    recv_sem=recv_sems.at[d - 1, b],
                    device_id=(lax.rem(my + d, N_DEV),),
                    device_id_type=pl.DeviceIdType.MESH,
                )
                rdma.start()
                wave.append(rdma)
            return wave

        attn_batch(0)

        barrier_sem = pltpu.get_barrier_semaphore()
        for d in range(1, N_DEV):
            pl.semaphore_signal(
                barrier_sem, inc=1,
                device_id=(lax.rem(my + d, N_DEV),),
                device_id_type=pl.DeviceIdType.MESH,
            )
        pl.semaphore_wait(barrier_sem, N_DEV - 1)

        wave0 = send_wave(0)
        attn_batch(1)
        wave1 = send_wave(1)

        cp_wo.wait()
        wo_my = wo_v[pl.ds(my * CHUNK, CHUNK), :]
        acc = jnp.dot(mychunk_ref[...], wo_my,
                      preferred_element_type=jnp.float32)

        for d in range(1, N_DEV):
            wave0[d - 1].wait_recv()
            wave1[d - 1].wait_recv()
            origin = lax.rem(my - d + N_DEV, N_DEV)
            wo_o = wo_v[pl.ds(origin * CHUNK, CHUNK), :]
            acc = acc + jnp.dot(gather_ref[origin], wo_o,
                                preferred_element_type=jnp.float32)
        out_ref[...] = acc.astype(jnp.bfloat16).reshape(B, SQ, D_MODEL)

        for d in range(1, N_DEV):
            wave0[d - 1].wait_send()
            wave1[d - 1].wait_send()

    return pl.pallas_call(
        body,
        out_shape=jax.ShapeDtypeStruct((B, SQ, D_MODEL), jnp.bfloat16),
        in_specs=[
            pl.BlockSpec(memory_space=pltpu.MemorySpace.HBM),
            pl.BlockSpec(memory_space=pltpu.MemorySpace.HBM),
            pl.BlockSpec(memory_space=pltpu.MemorySpace.HBM),
            pl.BlockSpec(memory_space=pltpu.MemorySpace.HBM),
            pl.BlockSpec(memory_space=pltpu.MemorySpace.HBM),
        ],
        out_specs=pl.BlockSpec(memory_space=pltpu.VMEM),
        scratch_shapes=[
            pltpu.VMEM((B, SQ, D_MODEL), jnp.bfloat16),
            pltpu.VMEM((D_MODEL, CHUNK), jnp.bfloat16),
            pltpu.VMEM((B, SKV, HQ_LOCAL, DH), jnp.bfloat16),
            pltpu.VMEM((B, SKV, HQ_LOCAL, DH), jnp.bfloat16),
            pltpu.VMEM((HQ_LOCAL * DH * N_DEV, D_MODEL), jnp.bfloat16),
            pltpu.VMEM((ROWS, CHUNK), jnp.bfloat16),
            pltpu.VMEM((N_DEV, ROWS, CHUNK), jnp.bfloat16),
            pltpu.SemaphoreType.DMA((5,)),
            pltpu.SemaphoreType.DMA((N_DEV - 1, B)),
            pltpu.SemaphoreType.DMA((N_DEV - 1, B)),
        ],
        compiler_params=pltpu.CompilerParams(collective_id=0),
    )(x_b, wq_b, k_b, v_b, wo_b)
